# Initial kernel scaffold; baseline (speedup 1.0000x reference)
import jax
import jax.numpy as jnp
from jax.experimental import pallas as pl


def kernel(origin, process, roast_level, variety, flavors, target_finish_temp, altitude, bean_density, origin_table, process_table, roast_table, variety_table, flavor_W, flavor_b, cont_W, cont_b):
    raise NotImplementedError("write your pallas kernel here")


# speedup vs baseline: 1.0149x; 1.0149x over previous
import functools

import jax
import jax.numpy as jnp
from jax import lax
from jax.experimental import pallas as pl
from jax.experimental.pallas import tpu as pltpu
from jax.experimental.pallas import tpu_sc as plsc

EMBED = 32
IDX_CHUNK = 128


@functools.lru_cache(maxsize=None)
def _make_gather(B: int, E: int):
    info = plsc.get_sparse_core_info()
    nc, ns = info.num_cores, info.num_subcores
    nw = nc * ns
    b_per_w = B // nw
    assert B % (8 * nw) == 0
    n_chunks = b_per_w // IDX_CHUNK
    assert b_per_w % IDX_CHUNK == 0

    mesh = plsc.VectorSubcoreMesh(core_axis_name="c", subcore_axis_name="s")

    @functools.partial(
        pl.kernel,
        mesh=mesh,
        out_type=jax.ShapeDtypeStruct((4, B, E), jnp.float32),
        scratch_types=[
            pltpu.VMEM((b_per_w,), jnp.int32),
            pltpu.VMEM((b_per_w, E), jnp.float32),
            pltpu.SemaphoreType.DMA,
        ],
        compiler_params=pltpu.CompilerParams(use_tc_tiling_on_sc=False),
    )
    def gather_k(oi, pi, ri, vi, ot, pt, rt, vt, out, idx_v, rows_v, sem):
        wid = lax.axis_index("s") * nc + lax.axis_index("c")
        base = wid * b_per_w
        for t, (ih, th) in enumerate(((oi, ot), (pi, pt), (ri, rt), (vi, vt))):
            pltpu.sync_copy(ih.at[pl.ds(base, b_per_w)], idx_v)
            cps = []
            for j in range(n_chunks):
                cps.append(
                    pltpu.async_copy(
                        th.at[idx_v.at[pl.ds(j * IDX_CHUNK, IDX_CHUNK)]],
                        rows_v.at[pl.ds(j * IDX_CHUNK, IDX_CHUNK)],
                        sem,
                    )
                )
            for cp in cps:
                cp.wait()
            pltpu.sync_copy(rows_v, out.at[t].at[pl.ds(base, b_per_w)])

    return gather_k


def _assemble_body(g_ref, fl_ref, cf_ref, fw_ref, fb_ref, cw_ref, cb_ref, out_ref):
    fl = (
        jnp.dot(fl_ref[...], fw_ref[...], preferred_element_type=jnp.float32)
        + fb_ref[...]
    )
    ct = (
        jnp.dot(cf_ref[...], cw_ref[...], preferred_element_type=jnp.float32)
        + cb_ref[...]
    )
    out_ref[...] = jnp.concatenate(
        [g_ref[0], g_ref[1], g_ref[2], g_ref[3], fl, ct], axis=1
    )


@functools.lru_cache(maxsize=None)
def _make_assemble(B: int, E: int, F: int, BM: int):
    return pl.pallas_call(
        _assemble_body,
        grid=(B // BM,),
        in_specs=[
            pl.BlockSpec((4, BM, E), lambda i: (0, i, 0)),
            pl.BlockSpec((BM, F), lambda i: (i, 0)),
            pl.BlockSpec((BM, 3), lambda i: (i, 0)),
            pl.BlockSpec((F, E), lambda i: (0, 0)),
            pl.BlockSpec((1, E), lambda i: (0, 0)),
            pl.BlockSpec((3, E), lambda i: (0, 0)),
            pl.BlockSpec((1, E), lambda i: (0, 0)),
        ],
        out_specs=pl.BlockSpec((BM, 6 * E), lambda i: (i, 0)),
        out_shape=jax.ShapeDtypeStruct((B, 6 * E), jnp.float32),
    )


def kernel(
    origin,
    process,
    roast_level,
    variety,
    flavors,
    target_finish_temp,
    altitude,
    bean_density,
    origin_table,
    process_table,
    roast_table,
    variety_table,
    flavor_W,
    flavor_b,
    cont_W,
    cont_b,
):
    B, F = flavors.shape
    E = origin_table.shape[1]
    oi = origin.reshape(B).astype(jnp.int32)
    pi = process.reshape(B).astype(jnp.int32)
    ri = roast_level.reshape(B).astype(jnp.int32)
    vi = variety.reshape(B).astype(jnp.int32)
    g = _make_gather(B, E)(
        oi, pi, ri, vi, origin_table, process_table, roast_table, variety_table
    )
    cf = jnp.concatenate([target_finish_temp, altitude, bean_density], axis=1)
    return _make_assemble(B, E, F, 1024)(
        g,
        flavors,
        cf,
        flavor_W,
        flavor_b.reshape(1, E),
        cont_W,
        cont_b.reshape(1, E),
    )


# speedup vs baseline: 1.0672x; 1.0516x over previous
import functools

import jax
import jax.numpy as jnp
from jax import lax
from jax.experimental import pallas as pl
from jax.experimental.pallas import tpu as pltpu
from jax.experimental.pallas import tpu_sc as plsc

IDX_CHUNK = 128


@functools.lru_cache(maxsize=None)
def _make_gather(B: int, E: int):
    info = plsc.get_sparse_core_info()
    nc, ns = info.num_cores, info.num_subcores
    nw = nc * ns
    b_per_w = B // nw
    assert B % (8 * nw) == 0
    n_chunks = b_per_w // IDX_CHUNK
    assert b_per_w % IDX_CHUNK == 0

    mesh = plsc.VectorSubcoreMesh(core_axis_name="c", subcore_axis_name="s")

    @functools.partial(
        pl.kernel,
        mesh=mesh,
        out_type=jax.ShapeDtypeStruct((4, B, E), jnp.float32),
        scratch_types=[
            pltpu.VMEM((b_per_w,), jnp.int32),
            pltpu.VMEM((b_per_w, E), jnp.float32),
            pltpu.SemaphoreType.DMA,
        ],
        compiler_params=pltpu.CompilerParams(use_tc_tiling_on_sc=False),
    )
    def gather_k(oi, pi, ri, vi, ot, pt, rt, vt, out, idx_v, rows_v, sem):
        wid = lax.axis_index("s") * nc + lax.axis_index("c")
        base = wid * b_per_w
        for t, (ih, th) in enumerate(((oi, ot), (pi, pt), (ri, rt), (vi, vt))):
            pltpu.sync_copy(ih.at[pl.ds(base, b_per_w)], idx_v)
            cps = []
            for j in range(n_chunks):
                cps.append(
                    pltpu.async_copy(
                        th.at[idx_v.at[pl.ds(j * IDX_CHUNK, IDX_CHUNK)]],
                        rows_v.at[pl.ds(j * IDX_CHUNK, IDX_CHUNK)],
                        sem,
                    )
                )
            for cp in cps:
                cp.wait()
            pltpu.sync_copy(rows_v, out.at[t].at[pl.ds(base, b_per_w)])

    return gather_k


def _make_assemble_body(BM, E):
    BMq = BM // 4

    def body(g_ref, fa_ref, cf_ref, fw_ref, fb_ref, cw_ref, cb_ref, out_ref):
        flp = (
            jnp.dot(fa_ref[...], fw_ref[...], preferred_element_type=jnp.float32)
            + fb_ref[...]
        )
        ctp = (
            jnp.dot(cf_ref[...], cw_ref[...], preferred_element_type=jnp.float32)
            + cb_ref[...]
        )
        gs = [g_ref[t] for t in range(4)]
        rows = []
        for r in range(4):
            lo, hi = 32 * r, 32 * r + 32
            rows.append(
                jnp.concatenate(
                    [
                        gs[0][:, lo:hi],
                        gs[1][:, lo:hi],
                        gs[2][:, lo:hi],
                        gs[3][:, lo:hi],
                        flp[:, lo:hi],
                        ctp[:, lo:hi],
                    ],
                    axis=1,
                )
            )
        out_ref[...] = jnp.stack(rows, axis=1).reshape(BM, 6 * E)

    return body


@functools.lru_cache(maxsize=None)
def _make_assemble(B: int, E: int, F: int, BM: int):
    BMq = BM // 4
    return pl.pallas_call(
        _make_assemble_body(BM, E),
        grid=(B // BM,),
        in_specs=[
            pl.BlockSpec((4, BMq, 4 * E), lambda i: (0, i, 0)),
            pl.BlockSpec((BMq, 4 * F), lambda i: (i, 0)),
            pl.BlockSpec((BMq, 12), lambda i: (i, 0)),
            pl.BlockSpec((4 * F, 4 * E), lambda i: (0, 0)),
            pl.BlockSpec((1, 4 * E), lambda i: (0, 0)),
            pl.BlockSpec((12, 4 * E), lambda i: (0, 0)),
            pl.BlockSpec((1, 4 * E), lambda i: (0, 0)),
        ],
        out_specs=pl.BlockSpec((BM, 6 * E), lambda i: (i, 0)),
        out_shape=jax.ShapeDtypeStruct((B, 6 * E), jnp.float32),
    )


def kernel(
    origin,
    process,
    roast_level,
    variety,
    flavors,
    target_finish_temp,
    altitude,
    bean_density,
    origin_table,
    process_table,
    roast_table,
    variety_table,
    flavor_W,
    flavor_b,
    cont_W,
    cont_b,
):
    B, F = flavors.shape
    E = origin_table.shape[1]
    oi = origin.reshape(B).astype(jnp.int32)
    pi = process.reshape(B).astype(jnp.int32)
    ri = roast_level.reshape(B).astype(jnp.int32)
    vi = variety.reshape(B).astype(jnp.int32)
    g = _make_gather(B, E)(
        oi, pi, ri, vi, origin_table, process_table, roast_table, variety_table
    )
    g = jnp.reshape(g, (4, B // 4, 4 * E))
    faP = jnp.reshape(flavors, (B // 4, 4 * F))
    cfP = jnp.reshape(
        jnp.concatenate([target_finish_temp, altitude, bean_density], axis=1),
        (B // 4, 12),
    )
    eye4 = jnp.eye(4, dtype=jnp.float32)
    fwBD = jnp.kron(eye4, flavor_W)
    cwBD = jnp.kron(eye4, cont_W)
    fbP = jnp.tile(flavor_b, 4).reshape(1, 4 * E)
    cbP = jnp.tile(cont_b, 4).reshape(1, 4 * E)
    return _make_assemble(B, E, F, 1024)(g, faP, cfP, fwBD, fbP, cwBD, cbP)


# speedup vs baseline: 1.3194x; 1.2363x over previous
import functools

import jax
import jax.numpy as jnp
from jax import lax
from jax.experimental import pallas as pl
from jax.experimental.pallas import tpu as pltpu
from jax.experimental.pallas import tpu_sc as plsc

IDX_CHUNK = 128


@functools.lru_cache(maxsize=None)
def _make_gather(B: int, E: int):
    info = plsc.get_sparse_core_info()
    nc, ns = info.num_cores, info.num_subcores
    nw = nc * ns
    b_per_w = B // nw
    assert B % (8 * nw) == 0
    n_chunks = b_per_w // IDX_CHUNK
    assert b_per_w % IDX_CHUNK == 0

    mesh = plsc.VectorSubcoreMesh(core_axis_name="c", subcore_axis_name="s")

    @functools.partial(
        pl.kernel,
        mesh=mesh,
        out_type=jax.ShapeDtypeStruct((4, B, E), jnp.float32),
        scratch_types=[
            pltpu.VMEM((b_per_w,), jnp.int32),
            pltpu.VMEM((b_per_w, E), jnp.float32),
            pltpu.SemaphoreType.DMA,
        ],
        compiler_params=pltpu.CompilerParams(use_tc_tiling_on_sc=False),
    )
    def gather_k(oi, pi, ri, vi, ot, pt, rt, vt, out, idx_v, rows_v, sem):
        wid = lax.axis_index("s") * nc + lax.axis_index("c")
        base = wid * b_per_w
        for t, (ih, th) in enumerate(((oi, ot), (pi, pt), (ri, rt), (vi, vt))):
            pltpu.sync_copy(ih.at[pl.ds(base, b_per_w)], idx_v)
            cps = []
            for j in range(n_chunks):
                cps.append(
                    pltpu.async_copy(
                        th.at[idx_v.at[pl.ds(j * IDX_CHUNK, IDX_CHUNK)]],
                        rows_v.at[pl.ds(j * IDX_CHUNK, IDX_CHUNK)],
                        sem,
                    )
                )
            for cp in cps:
                cp.wait()
            pltpu.sync_copy(rows_v, out.at[t].at[pl.ds(base, b_per_w)])

    return gather_k


def _make_assemble_body(BM, E):
    def body(g_ref, fa_ref, cf_ref, fw_ref, fb_ref, cw_ref, cb_ref, out_ref):
        gs = [g_ref[t] for t in range(4)]
        rows = []
        for r in range(4):
            lo, hi = E * r, E * r + E
            rows.append(jnp.concatenate([gs[t][:, lo:hi] for t in range(4)], axis=1))
        gblk = jnp.stack(rows, axis=1).reshape(BM, 4 * E)
        gT = jnp.transpose(gblk)
        flT = (
            lax.dot_general(
                fw_ref[...],
                fa_ref[...],
                (((0,), (1,)), ((), ())),
                preferred_element_type=jnp.float32,
            )
            + fb_ref[...]
        )
        ctT = (
            lax.dot_general(
                cw_ref[...],
                cf_ref[...],
                (((0,), (0,)), ((), ())),
                preferred_element_type=jnp.float32,
            )
            + cb_ref[...]
        )
        out_ref[...] = jnp.concatenate([gT, flT, ctT], axis=0)

    return body


@functools.lru_cache(maxsize=None)
def _make_assemble(B: int, E: int, F: int, BM: int):
    BMq = BM // 4
    return pl.pallas_call(
        _make_assemble_body(BM, E),
        grid=(B // BM,),
        in_specs=[
            pl.BlockSpec((4, BMq, 4 * E), lambda i: (0, i, 0)),
            pl.BlockSpec((BM, F), lambda i: (i, 0)),
            pl.BlockSpec((3, BM), lambda i: (0, i)),
            pl.BlockSpec((F, E), lambda i: (0, 0)),
            pl.BlockSpec((E, 1), lambda i: (0, 0)),
            pl.BlockSpec((3, E), lambda i: (0, 0)),
            pl.BlockSpec((E, 1), lambda i: (0, 0)),
        ],
        out_specs=pl.BlockSpec((6 * E, BM), lambda i: (0, i)),
        out_shape=jax.ShapeDtypeStruct((6 * E, B), jnp.float32),
    )


def kernel(
    origin,
    process,
    roast_level,
    variety,
    flavors,
    target_finish_temp,
    altitude,
    bean_density,
    origin_table,
    process_table,
    roast_table,
    variety_table,
    flavor_W,
    flavor_b,
    cont_W,
    cont_b,
):
    B, F = flavors.shape
    E = origin_table.shape[1]
    oi = origin.reshape(B).astype(jnp.int32)
    pi = process.reshape(B).astype(jnp.int32)
    ri = roast_level.reshape(B).astype(jnp.int32)
    vi = variety.reshape(B).astype(jnp.int32)
    g = _make_gather(B, E)(
        oi, pi, ri, vi, origin_table, process_table, roast_table, variety_table
    )
    g = jnp.reshape(g, (4, B // 4, 4 * E))
    cfT = jnp.concatenate(
        [target_finish_temp.T, altitude.T, bean_density.T], axis=0
    )
    outT = _make_assemble(B, E, F, 1024)(
        g,
        flavors,
        cfT,
        flavor_W,
        flavor_b.reshape(E, 1),
        cont_W,
        cont_b.reshape(E, 1),
    )
    return outT.T
